# baseline (speedup 1.0000x reference)
import functools

import jax
import jax.numpy as jnp
import numpy as np
from jax.experimental import pallas as pl
from jax.experimental.pallas import tpu as pltpu

_LOC_SCOPE = 3.0
_LOC_BIN_SIZE = 0.5
_NUM_HEAD_BIN = 12
_PRE_NMS_TOP_N = 9000
_POST_NMS_TOP_N = 512
_NMS_THRESH = 0.85
_MEAN_SIZES = np.array([[1.52563191, 1.62856739, 3.8831164],
                        [1.76255119, 0.66068622, 0.84422524],
                        [1.73698127, 0.59706367, 1.76282397]], dtype=np.float32)

_DEC_ROWS = 1024
_NMS_PAD = 9216
_NMS_COLS = 1152


def _decode_kernel(xyz_ref, reg_ref, out_ref):
    reg = reg_ref[0]
    xyz = xyz_ref[0]
    r = reg.shape[0]
    iota12 = jax.lax.broadcasted_iota(jnp.int32, (r, 12), 1)

    def argmax12(sl):
        m = jnp.max(sl, axis=1, keepdims=True)
        return jnp.min(jnp.where(sl == m, iota12, 12), axis=1, keepdims=True)

    def take12(sl, idx):
        return jnp.sum(jnp.where(iota12 == idx, sl, 0.0), axis=1, keepdims=True)

    xb = argmax12(reg[:, 0:12])
    zb = argmax12(reg[:, 12:24])
    px = xb.astype(jnp.float32) * _LOC_BIN_SIZE + _LOC_BIN_SIZE / 2.0 - _LOC_SCOPE
    pz = zb.astype(jnp.float32) * _LOC_BIN_SIZE + _LOC_BIN_SIZE / 2.0 - _LOC_SCOPE
    px = px + take12(reg[:, 24:36], xb) * _LOC_BIN_SIZE
    pz = pz + take12(reg[:, 36:48], zb) * _LOC_BIN_SIZE
    py0 = xyz[:, 1:2] + reg[:, 48:49]
    rb = argmax12(reg[:, 49:61])
    aper = 2.0 * np.pi / _NUM_HEAD_BIN
    rres = take12(reg[:, 61:73], rb) * (aper / 2.0)
    ry = rb.astype(jnp.float32) * aper + rres
    ry = jnp.mod(ry, 2.0 * np.pi)
    ry = jnp.where(ry > np.pi, ry - 2.0 * np.pi, ry)
    sr = reg[:, 73:76]
    px = px + xyz[:, 0:1]
    pz = pz + xyz[:, 2:3]
    out_ref[0] = jnp.concatenate(
        [px, py0, pz, sr, ry, jnp.zeros_like(ry)], axis=1)


def _decode(rpn_reg, xyz):
    b, n, _ = rpn_reg.shape
    return pl.pallas_call(
        _decode_kernel,
        grid=(b, n // _DEC_ROWS),
        in_specs=[
            pl.BlockSpec((1, _DEC_ROWS, 3), lambda i, j: (i, j, 0)),
            pl.BlockSpec((1, _DEC_ROWS, 76), lambda i, j: (i, j, 0)),
        ],
        out_specs=pl.BlockSpec((1, _DEC_ROWS, 8), lambda i, j: (i, j, 0)),
        out_shape=jax.ShapeDtypeStruct((b, n, 8), jnp.float32),
        compiler_params=pltpu.CompilerParams(
            dimension_semantics=("parallel", "parallel")),
    )(xyz, rpn_reg)


def _nms_kernel(planes_ref, out_ref):
    nb = planes_ref.shape[0]
    rank = (jax.lax.broadcasted_iota(jnp.int32, (8, _NMS_COLS), 0) * _NMS_COLS
            + jax.lax.broadcasted_iota(jnp.int32, (8, _NMS_COLS), 1))
    oh = [(jax.lax.broadcasted_iota(jnp.int32, (1, 8), 1) == c
           ).astype(jnp.float32) for c in range(8)]
    dead = jnp.int32(2 ** 30)

    geom = []
    for b in range(nb):
        def fld(i, b=b):
            return planes_ref[b, 8 * i:8 * i + 8, :]

        x, y, z = fld(0), fld(1), fld(2)
        h, w, l = fld(3), fld(4), fld(5)
        ry, sc = fld(6), fld(7)
        x1 = x - l * 0.5
        x2 = x + l * 0.5
        z1 = z - w * 0.5
        z2 = z + w * 0.5
        area = (x2 - x1) * (z2 - z1)
        geom.append((x, y, z, h, w, l, ry, sc, x1, x2, z1, z2, area))

    m0 = jnp.where(rank < _PRE_NMS_TOP_N, rank, dead)

    def body(t, ms):
        out = []
        for b in range(nb):
            m = ms[b]
            x, y, z, h, w, l, ry, sc, x1, x2, z1, z2, area = geom[b]
            r = jnp.min(m, keepdims=True)
            sel = rank == r

            def pick(v, sel=sel):
                return jnp.sum(jnp.where(sel, v, 0.0), keepdims=True)

            bx, bz, bw, bl = pick(x), pick(z), pick(w), pick(l)
            bx1 = bx - bl * 0.5
            bx2 = bx + bl * 0.5
            bz1 = bz - bw * 0.5
            bz2 = bz + bw * 0.5
            barea = (bx2 - bx1) * (bz2 - bz1)
            iw = jnp.maximum(jnp.minimum(bx2, x2) - jnp.maximum(bx1, x1), 0.0)
            iz = jnp.maximum(jnp.minimum(bz2, z2) - jnp.maximum(bz1, z1), 0.0)
            inter = iw * iz
            iou = inter / jnp.maximum(barea + area - inter, 1e-6)
            out.append(jnp.where((iou <= _NMS_THRESH) & (rank != r), m, dead))

            row = (bx * oh[0] + pick(y) * oh[1] + bz * oh[2]
                   + pick(h) * oh[3] + bw * oh[4] + bl * oh[5]
                   + pick(ry) * oh[6] + pick(sc) * oh[7])
            out_ref[b, pl.ds(t, 1), :] = row
        return tuple(out)

    jax.lax.fori_loop(0, _POST_NMS_TOP_N, body, (m0,) * nb)


def _nms(planes):
    b = planes.shape[0]
    return pl.pallas_call(
        _nms_kernel,
        out_shape=jax.ShapeDtypeStruct((b, _POST_NMS_TOP_N, 8), jnp.float32),
    )(planes)


@jax.jit
def kernel(rpn_cls, rpn_reg, xyz):
    b, n = rpn_cls.shape
    dec = _decode(rpn_reg, xyz)

    scores = jnp.tile(rpn_cls, (1, _MEAN_SIZES.shape[0]))
    sv, order = jax.lax.top_k(scores, _PRE_NMS_TOP_N)
    anchor = order // n
    pt = order % n

    cd = jnp.take_along_axis(dec, pt[..., None], axis=1)
    asz = jnp.asarray(_MEAN_SIZES)[anchor]
    px, py0, pz = cd[..., 0], cd[..., 1], cd[..., 2]
    h = cd[..., 3] * asz[..., 0] + asz[..., 0]
    w = cd[..., 4] * asz[..., 1] + asz[..., 1]
    l = cd[..., 5] * asz[..., 2] + asz[..., 2]
    y = py0 + h * 0.5
    ry = cd[..., 6]

    planes = jnp.stack([px, y, pz, h, w, l, ry, sv], axis=2)
    planes = jnp.pad(planes, ((0, 0), (0, _NMS_PAD - _PRE_NMS_TOP_N), (0, 0)))
    planes = planes.reshape(b, 8, _NMS_COLS, 8).transpose(0, 3, 1, 2)
    planes = planes.reshape(b, 64, _NMS_COLS)

    out = _nms(planes)
    return out[..., :7], out[..., 7]


# speedup vs baseline: 1.0535x; 1.0535x over previous
import functools

import jax
import jax.numpy as jnp
import numpy as np
from jax.experimental import pallas as pl
from jax.experimental.pallas import tpu as pltpu

_LOC_SCOPE = 3.0
_LOC_BIN_SIZE = 0.5
_NUM_HEAD_BIN = 12
_PRE_NMS_TOP_N = 9000
_POST_NMS_TOP_N = 512
_NMS_THRESH = 0.85
_MEAN_SIZES = np.array([[1.52563191, 1.62856739, 3.8831164],
                        [1.76255119, 0.66068622, 0.84422524],
                        [1.73698127, 0.59706367, 1.76282397]], dtype=np.float32)

_DEC_ROWS = 1024
_NMS_PAD = 9216
_NMS_COLS = 1152


def _decode_kernel(xyz_ref, reg_ref, out_ref):
    reg = reg_ref[0]
    xyz = xyz_ref[0]
    r = reg.shape[0]
    iota12 = jax.lax.broadcasted_iota(jnp.int32, (r, 12), 1)

    def argmax12(sl):
        m = jnp.max(sl, axis=1, keepdims=True)
        return jnp.min(jnp.where(sl == m, iota12, 12), axis=1, keepdims=True)

    def take12(sl, idx):
        return jnp.sum(jnp.where(iota12 == idx, sl, 0.0), axis=1, keepdims=True)

    xb = argmax12(reg[:, 0:12])
    zb = argmax12(reg[:, 12:24])
    px = xb.astype(jnp.float32) * _LOC_BIN_SIZE + _LOC_BIN_SIZE / 2.0 - _LOC_SCOPE
    pz = zb.astype(jnp.float32) * _LOC_BIN_SIZE + _LOC_BIN_SIZE / 2.0 - _LOC_SCOPE
    px = px + take12(reg[:, 24:36], xb) * _LOC_BIN_SIZE
    pz = pz + take12(reg[:, 36:48], zb) * _LOC_BIN_SIZE
    py0 = xyz[:, 1:2] + reg[:, 48:49]
    rb = argmax12(reg[:, 49:61])
    aper = 2.0 * np.pi / _NUM_HEAD_BIN
    rres = take12(reg[:, 61:73], rb) * (aper / 2.0)
    ry = rb.astype(jnp.float32) * aper + rres
    ry = jnp.mod(ry, 2.0 * np.pi)
    ry = jnp.where(ry > np.pi, ry - 2.0 * np.pi, ry)
    sr = reg[:, 73:76]
    px = px + xyz[:, 0:1]
    pz = pz + xyz[:, 2:3]
    out_ref[0] = jnp.concatenate(
        [px, py0, pz, sr, ry, jnp.zeros_like(ry)], axis=1)


def _decode(rpn_reg, xyz):
    b, n, _ = rpn_reg.shape
    return pl.pallas_call(
        _decode_kernel,
        grid=(b, n // _DEC_ROWS),
        in_specs=[
            pl.BlockSpec((1, _DEC_ROWS, 3), lambda i, j: (i, j, 0)),
            pl.BlockSpec((1, _DEC_ROWS, 76), lambda i, j: (i, j, 0)),
        ],
        out_specs=pl.BlockSpec((1, _DEC_ROWS, 8), lambda i, j: (i, j, 0)),
        out_shape=jax.ShapeDtypeStruct((b, n, 8), jnp.float32),
        compiler_params=pltpu.CompilerParams(
            dimension_semantics=("parallel", "parallel")),
    )(xyz, rpn_reg)


def _nms_kernel(planes_ref, out_ref):
    nb = planes_ref.shape[0]
    rank = (jax.lax.broadcasted_iota(jnp.int32, (8, _NMS_COLS), 0) * _NMS_COLS
            + jax.lax.broadcasted_iota(jnp.int32, (8, _NMS_COLS), 1))
    oh = [(jax.lax.broadcasted_iota(jnp.int32, (1, 8), 1) == c
           ).astype(jnp.float32) for c in range(8)]
    dead = jnp.int32(2 ** 30)

    geom = []
    for b in range(nb):
        def fld(i, b=b):
            return planes_ref[b, 8 * i:8 * i + 8, :]

        x, y, z = fld(0), fld(1), fld(2)
        h, w, l = fld(3), fld(4), fld(5)
        ry, sc = fld(6), fld(7)
        x1 = x - l * 0.5
        x2 = x + l * 0.5
        z1 = z - w * 0.5
        z2 = z + w * 0.5
        area = (x2 - x1) * (z2 - z1)
        geom.append((x, y, z, h, w, l, ry, sc, x1, x2, z1, z2, area))

    m0 = jnp.where(rank < _PRE_NMS_TOP_N, rank, dead)

    def body(t, ms):
        out = []
        for b in range(nb):
            m = ms[b]
            x, y, z, h, w, l, ry, sc, x1, x2, z1, z2, area = geom[b]
            r = jnp.min(m, keepdims=True)
            sel = rank == r

            def pick(v, sel=sel):
                return jnp.sum(jnp.where(sel, v, 0.0), keepdims=True)

            bx, bz, bw, bl = pick(x), pick(z), pick(w), pick(l)
            bx1 = bx - bl * 0.5
            bx2 = bx + bl * 0.5
            bz1 = bz - bw * 0.5
            bz2 = bz + bw * 0.5
            barea = (bx2 - bx1) * (bz2 - bz1)
            iw = jnp.maximum(jnp.minimum(bx2, x2) - jnp.maximum(bx1, x1), 0.0)
            iz = jnp.maximum(jnp.minimum(bz2, z2) - jnp.maximum(bz1, z1), 0.0)
            inter = iw * iz
            iou = inter / jnp.maximum(barea + area - inter, 1e-6)
            out.append(jnp.where((iou <= _NMS_THRESH) & (rank != r), m, dead))

            row = (bx * oh[0] + pick(y) * oh[1] + bz * oh[2]
                   + pick(h) * oh[3] + bw * oh[4] + bl * oh[5]
                   + pick(ry) * oh[6] + pick(sc) * oh[7])
            out_ref[b, pl.ds(t, 1), :] = row
        return tuple(out)

    jax.lax.fori_loop(0, _POST_NMS_TOP_N, body, (m0,) * nb, unroll=2)


def _nms(planes):
    b = planes.shape[0]
    return pl.pallas_call(
        _nms_kernel,
        out_shape=jax.ShapeDtypeStruct((b, _POST_NMS_TOP_N, 8), jnp.float32),
    )(planes)


@jax.jit
def kernel(rpn_cls, rpn_reg, xyz):
    b, n = rpn_cls.shape
    dec = _decode(rpn_reg, xyz)

    scores = jnp.tile(rpn_cls, (1, _MEAN_SIZES.shape[0]))
    sv, order = jax.lax.top_k(scores, _PRE_NMS_TOP_N)
    anchor = order // n
    pt = order % n

    cd = jnp.take_along_axis(dec, pt[..., None], axis=1)
    asz = jnp.asarray(_MEAN_SIZES)[anchor]
    px, py0, pz = cd[..., 0], cd[..., 1], cd[..., 2]
    h = cd[..., 3] * asz[..., 0] + asz[..., 0]
    w = cd[..., 4] * asz[..., 1] + asz[..., 1]
    l = cd[..., 5] * asz[..., 2] + asz[..., 2]
    y = py0 + h * 0.5
    ry = cd[..., 6]

    planes = jnp.stack([px, y, pz, h, w, l, ry, sv], axis=2)
    planes = jnp.pad(planes, ((0, 0), (0, _NMS_PAD - _PRE_NMS_TOP_N), (0, 0)))
    planes = planes.reshape(b, 8, _NMS_COLS, 8).transpose(0, 3, 1, 2)
    planes = planes.reshape(b, 64, _NMS_COLS)

    out = _nms(planes)
    return out[..., :7], out[..., 7]


# speedup vs baseline: 1.0866x; 1.0314x over previous
import functools

import jax
import jax.numpy as jnp
import numpy as np
from jax.experimental import pallas as pl
from jax.experimental.pallas import tpu as pltpu

_LOC_SCOPE = 3.0
_LOC_BIN_SIZE = 0.5
_NUM_HEAD_BIN = 12
_PRE_NMS_TOP_N = 9000
_POST_NMS_TOP_N = 512
_NMS_THRESH = 0.85
_MEAN_SIZES = np.array([[1.52563191, 1.62856739, 3.8831164],
                        [1.76255119, 0.66068622, 0.84422524],
                        [1.73698127, 0.59706367, 1.76282397]], dtype=np.float32)

_DEC_ROWS = 1024
_NMS_PAD = 9216
_NMS_COLS = 1152


def _decode_kernel(xyz_ref, reg_ref, out_ref):
    reg = reg_ref[0]
    xyz = xyz_ref[0]
    r = reg.shape[0]
    iota12 = jax.lax.broadcasted_iota(jnp.int32, (r, 12), 1)

    def argmax12(sl):
        m = jnp.max(sl, axis=1, keepdims=True)
        return jnp.min(jnp.where(sl == m, iota12, 12), axis=1, keepdims=True)

    def take12(sl, idx):
        return jnp.sum(jnp.where(iota12 == idx, sl, 0.0), axis=1, keepdims=True)

    xb = argmax12(reg[:, 0:12])
    zb = argmax12(reg[:, 12:24])
    px = xb.astype(jnp.float32) * _LOC_BIN_SIZE + _LOC_BIN_SIZE / 2.0 - _LOC_SCOPE
    pz = zb.astype(jnp.float32) * _LOC_BIN_SIZE + _LOC_BIN_SIZE / 2.0 - _LOC_SCOPE
    px = px + take12(reg[:, 24:36], xb) * _LOC_BIN_SIZE
    pz = pz + take12(reg[:, 36:48], zb) * _LOC_BIN_SIZE
    py0 = xyz[:, 1:2] + reg[:, 48:49]
    rb = argmax12(reg[:, 49:61])
    aper = 2.0 * np.pi / _NUM_HEAD_BIN
    rres = take12(reg[:, 61:73], rb) * (aper / 2.0)
    ry = rb.astype(jnp.float32) * aper + rres
    ry = jnp.mod(ry, 2.0 * np.pi)
    ry = jnp.where(ry > np.pi, ry - 2.0 * np.pi, ry)
    sr = reg[:, 73:76]
    px = px + xyz[:, 0:1]
    pz = pz + xyz[:, 2:3]
    out_ref[0] = jnp.concatenate(
        [px, py0, pz, sr, ry, jnp.zeros_like(ry)], axis=1)


def _decode(rpn_reg, xyz):
    b, n, _ = rpn_reg.shape
    return pl.pallas_call(
        _decode_kernel,
        grid=(b, n // _DEC_ROWS),
        in_specs=[
            pl.BlockSpec((1, _DEC_ROWS, 3), lambda i, j: (i, j, 0)),
            pl.BlockSpec((1, _DEC_ROWS, 76), lambda i, j: (i, j, 0)),
        ],
        out_specs=pl.BlockSpec((1, _DEC_ROWS, 8), lambda i, j: (i, j, 0)),
        out_shape=jax.ShapeDtypeStruct((b, n, 8), jnp.float32),
        compiler_params=pltpu.CompilerParams(
            dimension_semantics=("parallel", "parallel")),
    )(xyz, rpn_reg)


def _nms_kernel(planes_ref, out_ref):
    nb = planes_ref.shape[0]
    rank = (jax.lax.broadcasted_iota(jnp.int32, (8, _NMS_COLS), 0) * _NMS_COLS
            + jax.lax.broadcasted_iota(jnp.int32, (8, _NMS_COLS), 1))
    oh = [(jax.lax.broadcasted_iota(jnp.int32, (1, 8), 1) == c
           ).astype(jnp.float32) for c in range(8)]
    dead = jnp.int32(2 ** 30)

    geom = []
    for b in range(nb):
        def fld(i, b=b):
            return planes_ref[b, 8 * i:8 * i + 8, :]

        x, y, z = fld(0), fld(1), fld(2)
        h, w, l = fld(3), fld(4), fld(5)
        ry, sc = fld(6), fld(7)
        x1 = x - l * 0.5
        x2 = x + l * 0.5
        z1 = z - w * 0.5
        z2 = z + w * 0.5
        area = (x2 - x1) * (z2 - z1)
        geom.append((x, y, z, h, w, l, ry, sc, x1, x2, z1, z2, area))

    m0 = jnp.where(rank < _PRE_NMS_TOP_N, rank, dead)

    def body(t, ms):
        out = []
        for b in range(nb):
            m = ms[b]
            x, y, z, h, w, l, ry, sc, x1, x2, z1, z2, area = geom[b]
            r = jnp.min(m, keepdims=True)
            sel = rank == r

            def pick(v, sel=sel):
                return jnp.sum(jnp.where(sel, v, 0.0), keepdims=True)

            bx, bz, bw, bl = pick(x), pick(z), pick(w), pick(l)
            bx1 = bx - bl * 0.5
            bx2 = bx + bl * 0.5
            bz1 = bz - bw * 0.5
            bz2 = bz + bw * 0.5
            barea = (bx2 - bx1) * (bz2 - bz1)
            iw = jnp.maximum(jnp.minimum(bx2, x2) - jnp.maximum(bx1, x1), 0.0)
            iz = jnp.maximum(jnp.minimum(bz2, z2) - jnp.maximum(bz1, z1), 0.0)
            inter = iw * iz
            iou = inter / jnp.maximum(barea + area - inter, 1e-6)
            out.append(jnp.where((iou <= _NMS_THRESH) & (rank != r), m, dead))

            row = (bx * oh[0] + pick(y) * oh[1] + bz * oh[2]
                   + pick(h) * oh[3] + bw * oh[4] + bl * oh[5]
                   + pick(ry) * oh[6] + pick(sc) * oh[7])
            out_ref[b, pl.ds(t, 1), :] = row
        return tuple(out)

    jax.lax.fori_loop(0, _POST_NMS_TOP_N, body, (m0,) * nb, unroll=4)


def _nms(planes):
    b = planes.shape[0]
    return pl.pallas_call(
        _nms_kernel,
        out_shape=jax.ShapeDtypeStruct((b, _POST_NMS_TOP_N, 8), jnp.float32),
    )(planes)


@jax.jit
def kernel(rpn_cls, rpn_reg, xyz):
    b, n = rpn_cls.shape
    dec = _decode(rpn_reg, xyz)

    scores = jnp.tile(rpn_cls, (1, _MEAN_SIZES.shape[0]))
    sv, order = jax.lax.top_k(scores, _PRE_NMS_TOP_N)
    anchor = order // n
    pt = order % n

    cd = jnp.take_along_axis(dec, pt[..., None], axis=1)
    asz = jnp.asarray(_MEAN_SIZES)[anchor]
    px, py0, pz = cd[..., 0], cd[..., 1], cd[..., 2]
    h = cd[..., 3] * asz[..., 0] + asz[..., 0]
    w = cd[..., 4] * asz[..., 1] + asz[..., 1]
    l = cd[..., 5] * asz[..., 2] + asz[..., 2]
    y = py0 + h * 0.5
    ry = cd[..., 6]

    planes = jnp.stack([px, y, pz, h, w, l, ry, sv], axis=2)
    planes = jnp.pad(planes, ((0, 0), (0, _NMS_PAD - _PRE_NMS_TOP_N), (0, 0)))
    planes = planes.reshape(b, 8, _NMS_COLS, 8).transpose(0, 3, 1, 2)
    planes = planes.reshape(b, 64, _NMS_COLS)

    out = _nms(planes)
    return out[..., :7], out[..., 7]


# speedup vs baseline: 1.1043x; 1.0163x over previous
import functools

import jax
import jax.numpy as jnp
import numpy as np
from jax.experimental import pallas as pl
from jax.experimental.pallas import tpu as pltpu

_LOC_SCOPE = 3.0
_LOC_BIN_SIZE = 0.5
_NUM_HEAD_BIN = 12
_PRE_NMS_TOP_N = 9000
_POST_NMS_TOP_N = 512
_NMS_THRESH = 0.85
_MEAN_SIZES = np.array([[1.52563191, 1.62856739, 3.8831164],
                        [1.76255119, 0.66068622, 0.84422524],
                        [1.73698127, 0.59706367, 1.76282397]], dtype=np.float32)

_DEC_ROWS = 1024
_NMS_PAD = 9216
_NMS_COLS = 1152


def _decode_kernel(xyz_ref, reg_ref, out_ref):
    reg = reg_ref[0]
    xyz = xyz_ref[0]
    r = reg.shape[0]
    iota12 = jax.lax.broadcasted_iota(jnp.int32, (r, 12), 1)

    def argmax12(sl):
        m = jnp.max(sl, axis=1, keepdims=True)
        return jnp.min(jnp.where(sl == m, iota12, 12), axis=1, keepdims=True)

    def take12(sl, idx):
        return jnp.sum(jnp.where(iota12 == idx, sl, 0.0), axis=1, keepdims=True)

    xb = argmax12(reg[:, 0:12])
    zb = argmax12(reg[:, 12:24])
    px = xb.astype(jnp.float32) * _LOC_BIN_SIZE + _LOC_BIN_SIZE / 2.0 - _LOC_SCOPE
    pz = zb.astype(jnp.float32) * _LOC_BIN_SIZE + _LOC_BIN_SIZE / 2.0 - _LOC_SCOPE
    px = px + take12(reg[:, 24:36], xb) * _LOC_BIN_SIZE
    pz = pz + take12(reg[:, 36:48], zb) * _LOC_BIN_SIZE
    py0 = xyz[:, 1:2] + reg[:, 48:49]
    rb = argmax12(reg[:, 49:61])
    aper = 2.0 * np.pi / _NUM_HEAD_BIN
    rres = take12(reg[:, 61:73], rb) * (aper / 2.0)
    ry = rb.astype(jnp.float32) * aper + rres
    ry = jnp.mod(ry, 2.0 * np.pi)
    ry = jnp.where(ry > np.pi, ry - 2.0 * np.pi, ry)
    sr = reg[:, 73:76]
    px = px + xyz[:, 0:1]
    pz = pz + xyz[:, 2:3]
    out_ref[0] = jnp.concatenate(
        [px, py0, pz, sr, ry, jnp.zeros_like(ry)], axis=1)


def _decode(rpn_reg, xyz):
    b, n, _ = rpn_reg.shape
    return pl.pallas_call(
        _decode_kernel,
        grid=(b, n // _DEC_ROWS),
        in_specs=[
            pl.BlockSpec((1, _DEC_ROWS, 3), lambda i, j: (i, j, 0)),
            pl.BlockSpec((1, _DEC_ROWS, 76), lambda i, j: (i, j, 0)),
        ],
        out_specs=pl.BlockSpec((1, _DEC_ROWS, 8), lambda i, j: (i, j, 0)),
        out_shape=jax.ShapeDtypeStruct((b, n, 8), jnp.float32),
        compiler_params=pltpu.CompilerParams(
            dimension_semantics=("parallel", "parallel")),
    )(xyz, rpn_reg)


def _nms_kernel(planes_ref, out_ref):
    nb = planes_ref.shape[0]
    rank = (jax.lax.broadcasted_iota(jnp.int32, (8, _NMS_COLS), 0) * _NMS_COLS
            + jax.lax.broadcasted_iota(jnp.int32, (8, _NMS_COLS), 1))
    oh = [(jax.lax.broadcasted_iota(jnp.int32, (1, 8), 1) == c
           ).astype(jnp.float32) for c in range(8)]
    dead = jnp.int32(2 ** 30)

    geom = []
    for b in range(nb):
        def fld(i, b=b):
            return planes_ref[b, 8 * i:8 * i + 8, :]

        x, y, z = fld(0), fld(1), fld(2)
        h, w, l = fld(3), fld(4), fld(5)
        ry, sc = fld(6), fld(7)
        x1 = x - l * 0.5
        x2 = x + l * 0.5
        z1 = z - w * 0.5
        z2 = z + w * 0.5
        area = (x2 - x1) * (z2 - z1)
        geom.append((x, y, z, h, w, l, ry, sc, x1, x2, z1, z2, area))

    m0 = jnp.where(rank < _PRE_NMS_TOP_N, rank, dead)

    def body(t, ms):
        out = []
        for b in range(nb):
            m = ms[b]
            x, y, z, h, w, l, ry, sc, x1, x2, z1, z2, area = geom[b]
            r = jnp.min(m, keepdims=True)
            sel = rank == r

            def pick(v, sel=sel):
                return jnp.sum(jnp.where(sel, v, 0.0), keepdims=True)

            bx, bz, bw, bl = pick(x), pick(z), pick(w), pick(l)
            bx1 = bx - bl * 0.5
            bx2 = bx + bl * 0.5
            bz1 = bz - bw * 0.5
            bz2 = bz + bw * 0.5
            barea = (bx2 - bx1) * (bz2 - bz1)
            iw = jnp.maximum(jnp.minimum(bx2, x2) - jnp.maximum(bx1, x1), 0.0)
            iz = jnp.maximum(jnp.minimum(bz2, z2) - jnp.maximum(bz1, z1), 0.0)
            inter = iw * iz
            iou = inter / jnp.maximum(barea + area - inter, 1e-6)
            out.append(jnp.where((iou <= _NMS_THRESH) & (rank != r), m, dead))

            row = (bx * oh[0] + pick(y) * oh[1] + bz * oh[2]
                   + pick(h) * oh[3] + bw * oh[4] + bl * oh[5]
                   + pick(ry) * oh[6] + pick(sc) * oh[7])
            out_ref[b, pl.ds(t, 1), :] = row
        return tuple(out)

    jax.lax.fori_loop(0, _POST_NMS_TOP_N, body, (m0,) * nb, unroll=8)


def _nms(planes):
    b = planes.shape[0]
    return pl.pallas_call(
        _nms_kernel,
        out_shape=jax.ShapeDtypeStruct((b, _POST_NMS_TOP_N, 8), jnp.float32),
    )(planes)


@jax.jit
def kernel(rpn_cls, rpn_reg, xyz):
    b, n = rpn_cls.shape
    dec = _decode(rpn_reg, xyz)

    scores = jnp.tile(rpn_cls, (1, _MEAN_SIZES.shape[0]))
    sv, order = jax.lax.top_k(scores, _PRE_NMS_TOP_N)
    anchor = order // n
    pt = order % n

    cd = jnp.take_along_axis(dec, pt[..., None], axis=1)
    asz = jnp.asarray(_MEAN_SIZES)[anchor]
    px, py0, pz = cd[..., 0], cd[..., 1], cd[..., 2]
    h = cd[..., 3] * asz[..., 0] + asz[..., 0]
    w = cd[..., 4] * asz[..., 1] + asz[..., 1]
    l = cd[..., 5] * asz[..., 2] + asz[..., 2]
    y = py0 + h * 0.5
    ry = cd[..., 6]

    planes = jnp.stack([px, y, pz, h, w, l, ry, sv], axis=2)
    planes = jnp.pad(planes, ((0, 0), (0, _NMS_PAD - _PRE_NMS_TOP_N), (0, 0)))
    planes = planes.reshape(b, 8, _NMS_COLS, 8).transpose(0, 3, 1, 2)
    planes = planes.reshape(b, 64, _NMS_COLS)

    out = _nms(planes)
    return out[..., :7], out[..., 7]
